# baseline (speedup 1.0000x reference)
import functools

import jax
import jax.numpy as jnp
from jax import lax
from jax.experimental import pallas as pl
from jax.experimental.pallas import tpu as pltpu
from jax.experimental.pallas import tpu_sc as plsc

N_ATOMS = 100000
N_EDGES = 200000
D = 32
BD = 16
STEPS = 4

NC = 2
NS = 16
NW = NC * NS

CH = 128
E_PAD = 200704
EW = E_PAD // NW
G_SUP = 7
G_NSUP = EW // (G_SUP * CH)

ES = E_PAD // NS
S_NCH = ES // CH
S_INNER = 2
S_OUTER = S_NCH // S_INNER

HALF = N_ATOMS // 2
ACC_ROWS = 50048
PER_TILE = ACC_ROWS // NS


def _sc_mesh():
    return plsc.VectorSubcoreMesh(
        core_axis_name="c", subcore_axis_name="s", num_cores=NC, num_subcores=NS
    )


_SC_PARAMS = pltpu.CompilerParams(use_tc_tiling_on_sc=False)


def _make_gather(nch, sup):
    nsup = nch // sup
    ew = nch * CH
    e_half = ew * NW

    @functools.partial(
        pl.kernel,
        out_type=jax.ShapeDtypeStruct((e_half, D), jnp.float32),
        mesh=_sc_mesh(),
        scratch_types=[
            pltpu.VMEM((nch, CH), jnp.int32),
            pltpu.VMEM((2, sup * CH, D), jnp.float32),
            pltpu.SemaphoreType.DMA,
            pltpu.SemaphoreType.DMA,
        ],
        compiler_params=_SC_PARAMS,
    )
    def gather(h_hbm, idx_hbm, out_hbm, idx_v, buf_v, gsem, ssem):
        c = lax.axis_index("c")
        s = lax.axis_index("s")
        wid = s * NC + c
        pltpu.sync_copy(idx_hbm.at[wid], idx_v)

        stores = [None, None]
        for sp in range(nsup):
            slot = sp % 2
            if stores[slot] is not None:
                stores[slot].wait()
            descs = []
            for t in range(sup):
                d = pltpu.async_copy(
                    h_hbm.at[idx_v.at[sp * sup + t]],
                    buf_v.at[slot, pl.ds(t * CH, CH)],
                    gsem,
                )
                descs.append(d)
            for d in descs:
                d.wait()
            stores[slot] = pltpu.async_copy(
                buf_v.at[slot],
                out_hbm.at[pl.ds(wid * ew + sp * (sup * CH), sup * CH)],
                ssem,
            )
        for st in stores:
            st.wait()

    return gather


E_HALF_A = 102400
E_HALF_B = E_PAD - E_HALF_A
_gather_a = _make_gather(25, 5)
_gather_b = _make_gather(24, 6)


@functools.partial(
    pl.kernel,
    out_type=jax.ShapeDtypeStruct((N_ATOMS, D), jnp.float32),
    mesh=_sc_mesh(),
    scratch_types=[
        pltpu.VMEM((S_NCH, CH), jnp.int32),
        pltpu.VMEM((2, 2 * CH, D), jnp.float32),
        pltpu.VMEM_SHARED((ACC_ROWS, D), jnp.float32),
        pltpu.SemaphoreType.DMA,
        pltpu.SemaphoreType.DMA,
    ],
    compiler_params=_SC_PARAMS,
)
def _sc_scatter(val_hbm, aidx_hbm, zeros_hbm, out_hbm, idx_v, vbuf_v, acc,
                vsem0, vsem1):
    c = lax.axis_index("c")
    s = lax.axis_index("s")
    pltpu.sync_copy(zeros_hbm, acc.at[pl.ds(s * PER_TILE, PER_TILE)])
    pltpu.sync_copy(aidx_hbm.at[c, s], idx_v)
    plsc.subcore_barrier()

    ld0 = pltpu.async_copy(
        val_hbm.at[pl.ds(s * ES, 2 * CH)], vbuf_v.at[0], vsem0)

    def _scat(buf_slot, g):
        for t in range(2):
            pltpu.sync_copy(
                vbuf_v.at[buf_slot, pl.ds(t * CH, CH)],
                acc.at[idx_v.at[2 * g + t]],
                add=True,
            )

    @pl.loop(0, (S_NCH // 2 - 1) // 2)
    def _outer(o):
        g0 = 2 * o
        pltpu.async_copy(
            val_hbm.at[pl.ds(s * ES + (g0 + 1) * 2 * CH, 2 * CH)],
            vbuf_v.at[1], vsem1)
        pltpu.make_async_copy(
            val_hbm.at[pl.ds(0, 2 * CH)], vbuf_v.at[0], vsem0).wait()
        _scat(0, g0)
        pltpu.async_copy(
            val_hbm.at[pl.ds(s * ES + (g0 + 2) * 2 * CH, 2 * CH)],
            vbuf_v.at[0], vsem0)
        pltpu.make_async_copy(
            val_hbm.at[pl.ds(0, 2 * CH)], vbuf_v.at[1], vsem1).wait()
        _scat(1, g0 + 1)

    pltpu.make_async_copy(
        val_hbm.at[pl.ds(0, 2 * CH)], vbuf_v.at[0], vsem0).wait()
    _scat(0, S_NCH // 2 - 1)

    plsc.subcore_barrier()
    last = s == NS - 1

    @pl.when(jnp.logical_not(last))
    def _copy_full():
        pltpu.sync_copy(
            acc.at[pl.ds(s * PER_TILE, PER_TILE)],
            out_hbm.at[pl.ds(c * HALF + s * PER_TILE, PER_TILE)],
        )

    @pl.when(last)
    def _copy_last():
        n_last = HALF - (NS - 1) * PER_TILE
        pltpu.sync_copy(
            acc.at[pl.ds((NS - 1) * PER_TILE, n_last)],
            out_hbm.at[pl.ds(c * HALF + (NS - 1) * PER_TILE, n_last)],
        )


EB = 4096


def _transform_body(bond_ref, nbr_ref, v_ref, s_ref, m_ref, r_ref, out_ref):
    y = jnp.dot(nbr_ref[...].astype(jnp.bfloat16),
                v_ref[...].astype(jnp.bfloat16),
                preferred_element_type=jnp.float32)
    be = jnp.dot(bond_ref[...].astype(jnp.bfloat16),
                 s_ref[...].astype(jnp.bfloat16),
                 preferred_element_type=jnp.float32) + m_ref[...]
    out_ref[...] = jnp.dot((be * y).astype(jnp.bfloat16),
                           r_ref[...].astype(jnp.bfloat16),
                           preferred_element_type=jnp.float32)


def _make_transform(n_edges):
    return pl.pallas_call(
        _transform_body,
        grid=(n_edges // EB,),
        in_specs=[
            pl.BlockSpec((EB, BD), lambda i: (i, 0)),
            pl.BlockSpec((EB, D), lambda i: (i, 0)),
            pl.BlockSpec((D, (BD + 1) * D), lambda i: (0, 0)),
            pl.BlockSpec((BD, (BD + 1) * D), lambda i: (0, 0)),
            pl.BlockSpec((1, (BD + 1) * D), lambda i: (0, 0)),
            pl.BlockSpec(((BD + 1) * D, D), lambda i: (0, 0)),
        ],
        out_specs=pl.BlockSpec((EB, D), lambda i: (i, 0)),
        out_shape=jax.ShapeDtypeStruct((n_edges, D), jnp.float32),
    )


_transform_a = _make_transform(E_HALF_A)
_transform_b = _make_transform(E_HALF_B)


PACK = 4
NR = N_ATOMS // PACK
NB = 5000


def _gru_body(agg_ref, h_ref, wri_ref, wrh_ref, wzi_ref, wzh_ref, wni_ref,
              wnh_ref, br_ref, bz_ref, bni_ref, bnh_ref, out_ref):
    agg = agg_ref[...]
    h = h_ref[...]
    dot = lambda a, w: jnp.dot(a, w[...], preferred_element_type=jnp.float32)
    r = jax.nn.sigmoid(dot(agg, wri_ref) + dot(h, wrh_ref) + br_ref[...])
    z = jax.nn.sigmoid(dot(agg, wzi_ref) + dot(h, wzh_ref) + bz_ref[...])
    n = jnp.tanh(dot(agg, wni_ref) + bni_ref[...]
                 + r * (dot(h, wnh_ref) + bnh_ref[...]))
    out_ref[...] = (1.0 - z) * n + z * h


_gru = pl.pallas_call(
    _gru_body,
    grid=(NR // NB,),
    in_specs=[pl.BlockSpec((NB, PACK * D), lambda i: (i, 0))] * 2
    + [pl.BlockSpec((PACK * D, PACK * D), lambda i: (0, 0))] * 6
    + [pl.BlockSpec((1, PACK * D), lambda i: (0, 0))] * 4,
    out_specs=pl.BlockSpec((NB, PACK * D), lambda i: (i, 0)),
    out_shape=jax.ShapeDtypeStruct((NR, PACK * D), jnp.float32),
)


def kernel(atom_features, bond_features, pair_indices, W_lin, b_lin,
           W_ih, W_hh, b_ih, b_hh):
    pad = E_PAD - N_EDGES
    src = pair_indices[:, 0]
    dst = pair_indices[:, 1]

    dst_pad = jnp.concatenate([dst, jnp.zeros((pad,), jnp.int32)])
    dst_blk_a = dst_pad[:E_HALF_A].reshape(NW, 25, CH)
    dst_blk_b = dst_pad[E_HALF_A:].reshape(NW, 24, CH)

    src_pad = jnp.concatenate([src, jnp.full((pad,), -1, jnp.int32)])
    local0 = src_pad
    local1 = src_pad - HALF
    adj0 = jnp.where((local0 >= 0) & (local0 < HALF), local0, HALF)
    adj1 = jnp.where((local1 >= 0) & (local1 < HALF), local1, HALF)
    aidx = jnp.stack([adj0, adj1]).reshape(NC, NS, S_NCH, CH)

    zeros = jnp.zeros((PER_TILE, D), jnp.float32)
    bond_pad = jnp.concatenate(
        [bond_features, jnp.zeros((pad, BD), jnp.float32)]
    )

    w3 = W_lin.reshape(D, D, BD)
    v_main = jnp.transpose(w3, (1, 2, 0)).reshape(D, BD * D)
    v_bias = b_lin.reshape(D, D).T
    v_mat = jnp.concatenate([v_main, v_bias], axis=1)

    eye_bd = jnp.eye(BD, dtype=jnp.float32)
    s_mat = jnp.concatenate(
        [jnp.kron(eye_bd, jnp.ones((1, D), jnp.float32)),
         jnp.zeros((BD, D), jnp.float32)], axis=1)
    m_mat = jnp.concatenate(
        [jnp.zeros((1, BD * D), jnp.float32),
         jnp.ones((1, D), jnp.float32)], axis=1)
    r_mat = jnp.tile(jnp.eye(D, dtype=jnp.float32), (BD + 1, 1))

    eye4 = jnp.eye(PACK, dtype=jnp.float32)
    kron4 = lambda w: jnp.kron(eye4, w)
    wri, wzi, wni = (kron4(W_ih[g * D:(g + 1) * D].T) for g in range(3))
    wrh, wzh, wnh = (kron4(W_hh[g * D:(g + 1) * D].T) for g in range(3))
    tile4 = lambda b: jnp.tile(b.reshape(1, D), (1, PACK))
    br = tile4(b_ih[0:D] + b_hh[0:D])
    bz = tile4(b_ih[D:2 * D] + b_hh[D:2 * D])
    bni = tile4(b_ih[2 * D:3 * D])
    bnh = tile4(b_hh[2 * D:3 * D])

    bond_a = bond_pad[:E_HALF_A]
    bond_b = bond_pad[E_HALF_A:]

    h = atom_features
    for _ in range(STEPS):
        nbr_a = _gather_a(h, dst_blk_a)
        nbr_b = _gather_b(h, dst_blk_b)
        msg_a = _transform_a(bond_a, nbr_a, v_mat, s_mat, m_mat, r_mat)
        msg_b = _transform_b(bond_b, nbr_b, v_mat, s_mat, m_mat, r_mat)
        msg = jnp.concatenate([msg_a, msg_b])
        agg = _sc_scatter(msg, aidx, zeros)
        h = _gru(agg.reshape(NR, PACK * D), h.reshape(NR, PACK * D),
                 wri, wrh, wzi, wzh, wni, wnh, br, bz, bni, bnh)
        h = h.reshape(N_ATOMS, D)
    return h


# speedup vs baseline: 1.1067x; 1.1067x over previous
import functools

import jax
import jax.numpy as jnp
from jax import lax
from jax.experimental import pallas as pl
from jax.experimental.pallas import tpu as pltpu
from jax.experimental.pallas import tpu_sc as plsc

N_ATOMS = 100000
N_EDGES = 200000
D = 32
BD = 16
STEPS = 4

NC = 2
NS = 16
NW = NC * NS

CH = 128
E_PAD = 200704
EW = E_PAD // NW
G_SUP = 7
G_NSUP = EW // (G_SUP * CH)

ES = E_PAD // NS
S_NCH = ES // CH
S_INNER = 2
S_OUTER = S_NCH // S_INNER

HALF = N_ATOMS // 2
ACC_ROWS = 50048
PER_TILE = ACC_ROWS // NS


def _sc_mesh():
    return plsc.VectorSubcoreMesh(
        core_axis_name="c", subcore_axis_name="s", num_cores=NC, num_subcores=NS
    )


_SC_PARAMS = pltpu.CompilerParams(use_tc_tiling_on_sc=False)


def _make_gather(nch, sup):
    nsup = nch // sup
    ew = nch * CH
    e_half = ew * NW

    @functools.partial(
        pl.kernel,
        out_type=jax.ShapeDtypeStruct((e_half, D), jnp.float32),
        mesh=_sc_mesh(),
        scratch_types=[
            pltpu.VMEM((nch, CH), jnp.int32),
            pltpu.VMEM((2, sup * CH, D), jnp.float32),
            pltpu.SemaphoreType.DMA,
            pltpu.SemaphoreType.DMA,
        ],
        compiler_params=_SC_PARAMS,
    )
    def gather(h_hbm, idx_hbm, out_hbm, idx_v, buf_v, gsem, ssem):
        c = lax.axis_index("c")
        s = lax.axis_index("s")
        wid = s * NC + c
        pltpu.sync_copy(idx_hbm.at[wid], idx_v)

        stores = [None, None]
        for sp in range(nsup):
            slot = sp % 2
            if stores[slot] is not None:
                stores[slot].wait()
            descs = []
            for t in range(sup):
                d = pltpu.async_copy(
                    h_hbm.at[idx_v.at[sp * sup + t]],
                    buf_v.at[slot, pl.ds(t * CH, CH)],
                    gsem,
                )
                descs.append(d)
            for d in descs:
                d.wait()
            stores[slot] = pltpu.async_copy(
                buf_v.at[slot],
                out_hbm.at[pl.ds(wid * ew + sp * (sup * CH), sup * CH)],
                ssem,
            )
        for st in stores:
            st.wait()

    return gather


_gather = _make_gather(S_NCH // 2, G_SUP)


@functools.partial(
    pl.kernel,
    out_type=jax.ShapeDtypeStruct((N_ATOMS, D), jnp.float32),
    mesh=_sc_mesh(),
    scratch_types=[
        pltpu.VMEM((S_NCH, CH), jnp.int32),
        pltpu.VMEM((2, 2 * CH, D), jnp.float32),
        pltpu.VMEM_SHARED((ACC_ROWS, D), jnp.float32),
        pltpu.SemaphoreType.DMA,
        pltpu.SemaphoreType.DMA,
    ],
    compiler_params=_SC_PARAMS,
)
def _sc_scatter(val_hbm, aidx_hbm, zeros_hbm, out_hbm, idx_v, vbuf_v, acc,
                vsem0, vsem1):
    c = lax.axis_index("c")
    s = lax.axis_index("s")
    pltpu.sync_copy(zeros_hbm, acc.at[pl.ds(s * PER_TILE, PER_TILE)])
    pltpu.sync_copy(aidx_hbm.at[c, s], idx_v)
    plsc.subcore_barrier()

    ld0 = pltpu.async_copy(
        val_hbm.at[pl.ds(s * ES, 2 * CH)], vbuf_v.at[0], vsem0)

    def _scat(buf_slot, g):
        for t in range(2):
            pltpu.sync_copy(
                vbuf_v.at[buf_slot, pl.ds(t * CH, CH)],
                acc.at[idx_v.at[2 * g + t]],
                add=True,
            )

    @pl.loop(0, (S_NCH // 2 - 1) // 2)
    def _outer(o):
        g0 = 2 * o
        pltpu.async_copy(
            val_hbm.at[pl.ds(s * ES + (g0 + 1) * 2 * CH, 2 * CH)],
            vbuf_v.at[1], vsem1)
        pltpu.make_async_copy(
            val_hbm.at[pl.ds(0, 2 * CH)], vbuf_v.at[0], vsem0).wait()
        _scat(0, g0)
        pltpu.async_copy(
            val_hbm.at[pl.ds(s * ES + (g0 + 2) * 2 * CH, 2 * CH)],
            vbuf_v.at[0], vsem0)
        pltpu.make_async_copy(
            val_hbm.at[pl.ds(0, 2 * CH)], vbuf_v.at[1], vsem1).wait()
        _scat(1, g0 + 1)

    pltpu.make_async_copy(
        val_hbm.at[pl.ds(0, 2 * CH)], vbuf_v.at[0], vsem0).wait()
    _scat(0, S_NCH // 2 - 1)

    plsc.subcore_barrier()
    last = s == NS - 1

    @pl.when(jnp.logical_not(last))
    def _copy_full():
        pltpu.sync_copy(
            acc.at[pl.ds(s * PER_TILE, PER_TILE)],
            out_hbm.at[pl.ds(c * HALF + s * PER_TILE, PER_TILE)],
        )

    @pl.when(last)
    def _copy_last():
        n_last = HALF - (NS - 1) * PER_TILE
        pltpu.sync_copy(
            acc.at[pl.ds((NS - 1) * PER_TILE, n_last)],
            out_hbm.at[pl.ds(c * HALF + (NS - 1) * PER_TILE, n_last)],
        )


EB = 4096


def _transform_body(bond_ref, nbr_ref, v_ref, s_ref, m_ref, r_ref, out_ref):
    y = jnp.dot(nbr_ref[...].astype(jnp.bfloat16),
                v_ref[...].astype(jnp.bfloat16),
                preferred_element_type=jnp.float32)
    be = jnp.dot(bond_ref[...].astype(jnp.bfloat16),
                 s_ref[...].astype(jnp.bfloat16),
                 preferred_element_type=jnp.float32) + m_ref[...]
    out_ref[...] = jnp.dot((be * y).astype(jnp.bfloat16),
                           r_ref[...].astype(jnp.bfloat16),
                           preferred_element_type=jnp.float32)


def _make_transform(n_edges):
    return pl.pallas_call(
        _transform_body,
        grid=(n_edges // EB,),
        in_specs=[
            pl.BlockSpec((EB, BD), lambda i: (i, 0)),
            pl.BlockSpec((EB, D), lambda i: (i, 0)),
            pl.BlockSpec((D, (BD + 1) * D), lambda i: (0, 0)),
            pl.BlockSpec((BD, (BD + 1) * D), lambda i: (0, 0)),
            pl.BlockSpec((1, (BD + 1) * D), lambda i: (0, 0)),
            pl.BlockSpec(((BD + 1) * D, D), lambda i: (0, 0)),
        ],
        out_specs=pl.BlockSpec((EB, D), lambda i: (i, 0)),
        out_shape=jax.ShapeDtypeStruct((n_edges, D), jnp.float32),
    )


_transform = _make_transform(E_PAD)


PACK = 4
NR = N_ATOMS // PACK
NB = 5000


def _gru_body(agg_ref, h_ref, wri_ref, wrh_ref, wzi_ref, wzh_ref, wni_ref,
              wnh_ref, br_ref, bz_ref, bni_ref, bnh_ref, out_ref):
    agg = agg_ref[...]
    h = h_ref[...]
    dot = lambda a, w: jnp.dot(a, w[...], preferred_element_type=jnp.float32)
    r = jax.nn.sigmoid(dot(agg, wri_ref) + dot(h, wrh_ref) + br_ref[...])
    z = jax.nn.sigmoid(dot(agg, wzi_ref) + dot(h, wzh_ref) + bz_ref[...])
    n = jnp.tanh(dot(agg, wni_ref) + bni_ref[...]
                 + r * (dot(h, wnh_ref) + bnh_ref[...]))
    out_ref[...] = (1.0 - z) * n + z * h


_gru = pl.pallas_call(
    _gru_body,
    grid=(NR // NB,),
    in_specs=[pl.BlockSpec((NB, PACK * D), lambda i: (i, 0))] * 2
    + [pl.BlockSpec((PACK * D, PACK * D), lambda i: (0, 0))] * 6
    + [pl.BlockSpec((1, PACK * D), lambda i: (0, 0))] * 4,
    out_specs=pl.BlockSpec((NB, PACK * D), lambda i: (i, 0)),
    out_shape=jax.ShapeDtypeStruct((NR, PACK * D), jnp.float32),
)


def kernel(atom_features, bond_features, pair_indices, W_lin, b_lin,
           W_ih, W_hh, b_ih, b_hh):
    pad = E_PAD - N_EDGES
    src = pair_indices[:, 0]
    dst = pair_indices[:, 1]

    dst_pad = jnp.concatenate([dst, jnp.zeros((pad,), jnp.int32)])
    dst_blk = dst_pad.reshape(NW, S_NCH // 2, CH)

    src_pad = jnp.concatenate([src, jnp.full((pad,), -1, jnp.int32)])
    local0 = src_pad
    local1 = src_pad - HALF
    adj0 = jnp.where((local0 >= 0) & (local0 < HALF), local0, HALF)
    adj1 = jnp.where((local1 >= 0) & (local1 < HALF), local1, HALF)
    aidx = jnp.stack([adj0, adj1]).reshape(NC, NS, S_NCH, CH)

    zeros = jnp.zeros((PER_TILE, D), jnp.float32)
    bond_pad = jnp.concatenate(
        [bond_features, jnp.zeros((pad, BD), jnp.float32)]
    )

    w3 = W_lin.reshape(D, D, BD)
    v_main = jnp.transpose(w3, (1, 2, 0)).reshape(D, BD * D)
    v_bias = b_lin.reshape(D, D).T
    v_mat = jnp.concatenate([v_main, v_bias], axis=1)

    eye_bd = jnp.eye(BD, dtype=jnp.float32)
    s_mat = jnp.concatenate(
        [jnp.kron(eye_bd, jnp.ones((1, D), jnp.float32)),
         jnp.zeros((BD, D), jnp.float32)], axis=1)
    m_mat = jnp.concatenate(
        [jnp.zeros((1, BD * D), jnp.float32),
         jnp.ones((1, D), jnp.float32)], axis=1)
    r_mat = jnp.tile(jnp.eye(D, dtype=jnp.float32), (BD + 1, 1))

    eye4 = jnp.eye(PACK, dtype=jnp.float32)
    kron4 = lambda w: jnp.kron(eye4, w)
    wri, wzi, wni = (kron4(W_ih[g * D:(g + 1) * D].T) for g in range(3))
    wrh, wzh, wnh = (kron4(W_hh[g * D:(g + 1) * D].T) for g in range(3))
    tile4 = lambda b: jnp.tile(b.reshape(1, D), (1, PACK))
    br = tile4(b_ih[0:D] + b_hh[0:D])
    bz = tile4(b_ih[D:2 * D] + b_hh[D:2 * D])
    bni = tile4(b_ih[2 * D:3 * D])
    bnh = tile4(b_hh[2 * D:3 * D])

    h = atom_features
    for _ in range(STEPS):
        nbr = _gather(h, dst_blk)
        msg = _transform(bond_pad, nbr, v_mat, s_mat, m_mat, r_mat)
        agg = _sc_scatter(msg, aidx, zeros)
        h = _gru(agg.reshape(NR, PACK * D), h.reshape(NR, PACK * D),
                 wri, wrh, wzi, wzh, wni, wnh, br, bz, bni, bnh)
        h = h.reshape(N_ATOMS, D)
    return h
